# baseline (speedup 1.0000x reference)
import functools

import jax
import jax.numpy as jnp
from jax import lax
from jax.experimental import pallas as pl
from jax.experimental.pallas import tpu as pltpu
from jax.experimental.pallas import tpu_sc as plsc


def _sc_info():
    try:
        info = plsc.get_sparse_core_info()
        return info.num_cores, info.num_subcores
    except Exception:
        return 2, 16


_GCHUNK = 128


def _sc_gather(table, idx2d, n_rows, d):
    nc, ns = _sc_info()
    nw = nc * ns
    chunks_total = n_rows // _GCHUNK
    chunks_per_w = chunks_total // nw
    rows_per_w = chunks_per_w * _GCHUNK

    mesh = plsc.VectorSubcoreMesh(core_axis_name="c", subcore_axis_name="s")

    @functools.partial(
        pl.kernel,
        out_type=jax.ShapeDtypeStruct((n_rows, d), jnp.float32),
        mesh=mesh,
        scratch_types=[
            pltpu.VMEM((chunks_per_w, _GCHUNK), jnp.int32),
            pltpu.VMEM((_GCHUNK, d), jnp.float32),
            pltpu.SemaphoreType.DMA,
        ],
        compiler_params=pltpu.CompilerParams(use_tc_tiling_on_sc=False),
    )
    def gather_kernel(table_hbm, idx_hbm, out_hbm, idx_v, rows_v, sem):
        wid = lax.axis_index("s") * nc + lax.axis_index("c")
        cbase = wid * chunks_per_w
        rbase = wid * rows_per_w
        pltpu.sync_copy(idx_hbm.at[pl.ds(cbase, chunks_per_w)], idx_v)

        def body(j, carry):
            pltpu.async_copy(table_hbm.at[idx_v.at[j]], rows_v, sem).wait()
            pltpu.sync_copy(rows_v, out_hbm.at[pl.ds(rbase + j * _GCHUNK, _GCHUNK)])
            return carry

        lax.fori_loop(0, chunks_per_w, body, 0)

    return gather_kernel(table, idx2d)


def _tc_mlp(flat, W1p, b1r, W2r, b2r):
    b_, k = flat.shape
    h = W1p.shape[1]
    bm = 1024

    def body(x_ref, w1_ref, b1_ref, w2_ref, b2_ref, o_ref):
        x = x_ref[...]
        hh = jnp.maximum(
            jnp.dot(x, w1_ref[...], preferred_element_type=jnp.float32)
            + b1_ref[...],
            0.0,
        )
        o_ref[...] = jnp.sum(hh * w2_ref[...], axis=1, keepdims=True) + b2_ref[...]

    return pl.pallas_call(
        body,
        grid=(b_ // bm,),
        in_specs=[
            pl.BlockSpec((bm, k), lambda i: (i, 0)),
            pl.BlockSpec((k, h), lambda i: (0, 0)),
            pl.BlockSpec((1, h), lambda i: (0, 0)),
            pl.BlockSpec((1, h), lambda i: (0, 0)),
            pl.BlockSpec((1, 1), lambda i: (0, 0)),
        ],
        out_specs=pl.BlockSpec((bm, 1), lambda i: (i, 0)),
        out_shape=jax.ShapeDtypeStruct((b_, 1), jnp.float32),
    )(flat, W1p, b1r, W2r, b2r)


def kernel(X, table, W1, b1, W2, b2):
    b_, f = X.shape
    v, d = table.shape
    h = W1.shape[1]

    fp = 32
    Xp = jnp.pad(X, ((0, 0), (0, fp - f)))
    n_rows = b_ * fp
    idx2d = Xp.reshape(n_rows // _GCHUNK, _GCHUNK)

    W1p = jnp.pad(W1.reshape(f, d, h), ((0, fp - f), (0, 0), (0, 0))).reshape(
        fp * d, h
    )

    rows = _sc_gather(table, idx2d, n_rows, d)
    flat = rows.reshape(b_, fp * d)
    return _tc_mlp(flat, W1p, b1.reshape(1, h), W2.reshape(1, h), b2.reshape(1, 1))


# speedup vs baseline: 1.3644x; 1.3644x over previous
import functools

import jax
import jax.numpy as jnp
from jax import lax
from jax.experimental import pallas as pl
from jax.experimental.pallas import tpu as pltpu
from jax.experimental.pallas import tpu_sc as plsc


def _sc_info():
    try:
        info = plsc.get_sparse_core_info()
        return info.num_cores, info.num_subcores
    except Exception:
        return 2, 16


_GCHUNK = 128


def _sc_format(tableT, tail_pk, v, d):
    nc, ns = _sc_info()
    nw = nc * ns
    nblk = v // _GCHUNK
    tail_rows = tail_pk.shape[0]
    per_w = (nblk + nw - 1) // nw
    out_rows = (v * d) // _GCHUNK

    mesh = plsc.VectorSubcoreMesh(core_axis_name="c", subcore_axis_name="s")

    @functools.partial(
        pl.kernel,
        out_type=jax.ShapeDtypeStruct((out_rows, _GCHUNK), jnp.float32),
        mesh=mesh,
        scratch_types=[
            pltpu.VMEM((d, _GCHUNK), jnp.float32),
            pltpu.VMEM((d, _GCHUNK), jnp.float32),
            pltpu.VMEM((tail_rows, _GCHUNK), jnp.float32),
        ],
        compiler_params=pltpu.CompilerParams(
            use_tc_tiling_on_sc=True, needs_layout_passes=False
        ),
    )
    def format_kernel(tableT_hbm, tail_hbm, out_hbm, tin, tout, ttail):
        wid = lax.axis_index("s") * nc + lax.axis_index("c")
        d_lo = lax.iota(jnp.int32, 16)
        d_hi = d_lo + 16

        def body(j, carry):
            g = wid + nw * j

            @pl.when(g < nblk)
            def _():
                pltpu.sync_copy(tableT_hbm.at[:, pl.ds(g * _GCHUNK, _GCHUNK)], tin)

                def row(r, carry2):
                    for k in range(8):
                        dvec = d_lo if (k % 2 == 0) else d_hi
                        cvec = jnp.zeros((16,), jnp.int32) + (4 * r + (k // 2))
                        vals = plsc.load_gather(tin, [dvec, cvec])
                        tout[r, pl.ds(16 * k, 16)] = vals
                    return carry2

                lax.fori_loop(0, d, row, 0)
                pltpu.sync_copy(tout, out_hbm.at[pl.ds(g * 32, d)])

            return carry

        lax.fori_loop(0, per_w, body, 0)

        @pl.when(wid == 0)
        def _():
            pltpu.sync_copy(tail_hbm, ttail)
            pltpu.sync_copy(ttail, out_hbm.at[pl.ds(nblk * 32, tail_rows)])

    return format_kernel(tableT, tail_pk)


def _sc_gather(table_lin, idx2d, n_rows, d):
    nc, ns = _sc_info()
    nw = nc * ns
    chunks_total = n_rows // _GCHUNK
    chunks_per_w = chunks_total // nw
    rows_per_w = chunks_per_w * _GCHUNK

    mesh = plsc.VectorSubcoreMesh(core_axis_name="c", subcore_axis_name="s")

    @functools.partial(
        pl.kernel,
        out_type=jax.ShapeDtypeStruct((n_rows, d), jnp.float32),
        mesh=mesh,
        scratch_types=[
            pltpu.VMEM((chunks_per_w, _GCHUNK), jnp.int32),
            pltpu.VMEM((_GCHUNK, d), jnp.float32),
            pltpu.SemaphoreType.DMA,
        ],
        compiler_params=pltpu.CompilerParams(use_tc_tiling_on_sc=False),
    )
    def gather_kernel(table_hbm, idx_hbm, out_hbm, idx_v, rows_v, sem):
        wid = lax.axis_index("s") * nc + lax.axis_index("c")
        cbase = wid * chunks_per_w
        rbase = wid * rows_per_w
        pltpu.sync_copy(idx_hbm.at[pl.ds(cbase, chunks_per_w)], idx_v)

        def body(j, carry):
            pltpu.async_copy(table_hbm.at[idx_v.at[j]], rows_v, sem).wait()
            pltpu.sync_copy(rows_v, out_hbm.at[pl.ds(rbase + j * _GCHUNK, _GCHUNK)])
            return carry

        lax.fori_loop(0, chunks_per_w, body, 0)

    return gather_kernel(table_lin, idx2d)


def _tc_mlp(flat, W1, b1r, W2r, b2r):
    b_, k = flat.shape
    h = W1.shape[1]
    bm = 1024

    def body(x_ref, w1_ref, b1_ref, w2_ref, b2_ref, o_ref):
        x = x_ref[...]
        hh = jnp.maximum(
            jnp.dot(x, w1_ref[...], preferred_element_type=jnp.float32)
            + b1_ref[...],
            0.0,
        )
        o_ref[...] = jnp.sum(hh * w2_ref[...], axis=1, keepdims=True) + b2_ref[...]

    return pl.pallas_call(
        body,
        grid=(b_ // bm,),
        in_specs=[
            pl.BlockSpec((bm, k), lambda i: (i, 0)),
            pl.BlockSpec((k, h), lambda i: (0, 0)),
            pl.BlockSpec((1, h), lambda i: (0, 0)),
            pl.BlockSpec((1, h), lambda i: (0, 0)),
            pl.BlockSpec((1, 1), lambda i: (0, 0)),
        ],
        out_specs=pl.BlockSpec((bm, 1), lambda i: (i, 0)),
        out_shape=jax.ShapeDtypeStruct((b_, 1), jnp.float32),
    )(flat, W1, b1r, W2r, b2r)


def kernel(X, table, W1, b1, W2, b2):
    b_, f = X.shape
    v, d = table.shape
    h = W1.shape[1]
    n_rows = b_ * f

    tail = v % _GCHUNK
    tail_pk = table[v - tail :, :].reshape(tail * d // _GCHUNK, _GCHUNK)
    table_pk = _sc_format(table.T, tail_pk, v, d)
    table_lin = table_pk.reshape(v, d)

    idx2d = X.reshape(n_rows // _GCHUNK, _GCHUNK)
    rows = _sc_gather(table_lin, idx2d, n_rows, d)
    flat = rows.reshape(b_, f * d)
    return _tc_mlp(flat, W1, b1.reshape(1, h), W2.reshape(1, h), b2.reshape(1, 1))


# speedup vs baseline: 1.7190x; 1.2599x over previous
import functools

import jax
import jax.numpy as jnp
from jax import lax
from jax.experimental import pallas as pl
from jax.experimental.pallas import tpu as pltpu
from jax.experimental.pallas import tpu_sc as plsc


def _sc_info():
    try:
        info = plsc.get_sparse_core_info()
        return info.num_cores, info.num_subcores
    except Exception:
        return 2, 16


_GCHUNK = 128


def _sc_format(tableT, tail_pk, v, d):
    nc, ns = _sc_info()
    nw = nc * ns
    nblk = v // _GCHUNK
    tail_rows = tail_pk.shape[0]
    out_rows = (v * d) // _GCHUNK

    m = 2
    sb_per_w = nblk // (nw * m)
    nleft = nblk - sb_per_w * nw * m
    scols = m * _GCHUNK
    srows = m * d

    mesh = plsc.VectorSubcoreMesh(core_axis_name="c", subcore_axis_name="s")

    @functools.partial(
        pl.kernel,
        out_type=jax.ShapeDtypeStruct((out_rows, _GCHUNK), jnp.float32),
        mesh=mesh,
        scratch_types=[
            pltpu.VMEM((d, scols), jnp.float32),
            pltpu.VMEM((d, scols), jnp.float32),
            pltpu.VMEM((srows, _GCHUNK), jnp.float32),
            pltpu.VMEM((srows, _GCHUNK), jnp.float32),
            pltpu.VMEM((tail_rows, _GCHUNK), jnp.float32),
            pltpu.SemaphoreType.DMA,
            pltpu.SemaphoreType.DMA,
            pltpu.SemaphoreType.DMA,
            pltpu.SemaphoreType.DMA,
        ],
        compiler_params=pltpu.CompilerParams(
            use_tc_tiling_on_sc=True, needs_layout_passes=False
        ),
    )
    def format_kernel(tableT_hbm, tail_hbm, out_hbm, tin0, tin1, tout0, tout1,
                      ttail, si0, si1, so0, so1):
        wid = lax.axis_index("s") * nc + lax.axis_index("c")
        sb0 = wid * sb_per_w
        d_lo = lax.iota(jnp.int32, 16)
        d_hi = d_lo + 16
        tins = (tin0, tin1)
        touts = (tout0, tout1)
        sis = (si0, si1)
        sos = (so0, so1)

        def src(j):
            return tableT_hbm.at[:, pl.ds((sb0 + j) * scols, scols)]

        def dst(j):
            return out_hbm.at[pl.ds((sb0 + j) * srows, srows)]

        def issue_in(j, p):
            pltpu.async_copy(src(j), tins[p], sis[p])

        def wait_in(j, p):
            pltpu.make_async_copy(src(j), tins[p], sis[p]).wait()

        def issue_out(j, p):
            pltpu.async_copy(touts[p], dst(j), sos[p])

        def wait_out(j, p):
            pltpu.make_async_copy(touts[p], dst(j), sos[p]).wait()

        def compute(tin, tout):
            def rowpair(rr, carry):
                b = rr // 16
                col0 = _GCHUNK * b + 4 * (2 * rr - 32 * b)
                for i in range(2):
                    for k in range(8):
                        dvec = d_lo if (k % 2 == 0) else d_hi
                        cvec = jnp.zeros((16,), jnp.int32) + (
                            col0 + 4 * i + (k // 2)
                        )
                        vals = plsc.load_gather(tin, [dvec, cvec])
                        tout[2 * rr + i, pl.ds(16 * k, 16)] = vals
                return carry

            lax.fori_loop(0, 16 * m, rowpair, 0)

        def steady(j, p, first):
            wait_in(j, p)
            if not first:
                wait_out(j - 2, p)
            compute(tins[p], touts[p])
            issue_out(j, p)
            if j + 2 < sb_per_w:
                issue_in(j + 2, p)

        issue_in(0, 0)
        issue_in(1, 1)
        steady(0, 0, True)
        steady(1, 1, True)

        def body(jj, carry):
            j = 2 + 2 * jj
            wait_in(j, 0)
            wait_out(j - 2, 0)
            compute(tin0, tout0)
            issue_out(j, 0)

            @pl.when(j + 2 < sb_per_w)
            def _():
                pltpu.async_copy(
                    tableT_hbm.at[:, pl.ds((sb0 + j + 2) * scols, scols)],
                    tin0, si0,
                )
            wait_in(j + 1, 1)
            wait_out(j - 1, 1)
            compute(tin1, tout1)
            issue_out(j + 1, 1)

            @pl.when(j + 3 < sb_per_w)
            def _():
                pltpu.async_copy(
                    tableT_hbm.at[:, pl.ds((sb0 + j + 3) * scols, scols)],
                    tin1, si1,
                )

            return carry

        if sb_per_w > 2:
            nst = (sb_per_w - 2) // 2
            lax.fori_loop(0, nst, body, 0)
            for j in range(2 + 2 * nst, sb_per_w):
                p = j % 2
                wait_in(j, p)
                wait_out(j - 2, p)
                compute(tins[p], touts[p])
                issue_out(j, p)
        wait_out(sb_per_w - 2, sb_per_w % 2)
        wait_out(sb_per_w - 1, (sb_per_w - 1) % 2)

        if nleft:
            @pl.when(wid < nleft)
            def _():
                g = nblk - nleft + wid
                pltpu.sync_copy(
                    tableT_hbm.at[:, pl.ds(g * _GCHUNK, _GCHUNK)],
                    tin0.at[:, pl.ds(0, _GCHUNK)],
                )
                def rowpair(rr, carry):
                    for i in range(2):
                        for k in range(8):
                            dvec = d_lo if (k % 2 == 0) else d_hi
                            cvec = jnp.zeros((16,), jnp.int32) + (
                                4 * (2 * rr + i) + (k // 2)
                            )
                            vals = plsc.load_gather(tin0, [dvec, cvec])
                            tout0[2 * rr + i, pl.ds(16 * k, 16)] = vals
                    return carry

                lax.fori_loop(0, 16, rowpair, 0)
                pltpu.sync_copy(
                    tout0.at[pl.ds(0, 32)], out_hbm.at[pl.ds(g * 32, 32)]
                )

        @pl.when(wid == 0)
        def _():
            pltpu.sync_copy(tail_hbm, ttail)
            pltpu.sync_copy(ttail, out_hbm.at[pl.ds(nblk * 32, tail_rows)])

    return format_kernel(tableT, tail_pk)


def _sc_gather(table_lin, idx2d, n_rows, d):
    nc, ns = _sc_info()
    nw = nc * ns
    chunks_total = n_rows // _GCHUNK
    chunks_per_w = chunks_total // nw
    rows_per_w = chunks_per_w * _GCHUNK

    mesh = plsc.VectorSubcoreMesh(core_axis_name="c", subcore_axis_name="s")

    @functools.partial(
        pl.kernel,
        out_type=jax.ShapeDtypeStruct((n_rows, d), jnp.float32),
        mesh=mesh,
        scratch_types=[
            pltpu.VMEM((chunks_per_w, _GCHUNK), jnp.int32),
            pltpu.VMEM((_GCHUNK, d), jnp.float32),
            pltpu.SemaphoreType.DMA,
        ],
        compiler_params=pltpu.CompilerParams(use_tc_tiling_on_sc=False),
    )
    def gather_kernel(table_hbm, idx_hbm, out_hbm, idx_v, rows_v, sem):
        wid = lax.axis_index("s") * nc + lax.axis_index("c")
        cbase = wid * chunks_per_w
        rbase = wid * rows_per_w
        pltpu.sync_copy(idx_hbm.at[pl.ds(cbase, chunks_per_w)], idx_v)

        def body(j, carry):
            pltpu.async_copy(table_hbm.at[idx_v.at[j]], rows_v, sem).wait()
            pltpu.sync_copy(rows_v, out_hbm.at[pl.ds(rbase + j * _GCHUNK, _GCHUNK)])
            return carry

        lax.fori_loop(0, chunks_per_w, body, 0)

    return gather_kernel(table_lin, idx2d)


def _tc_mlp(flat, W1, b1r, W2r, b2r):
    b_, k = flat.shape
    h = W1.shape[1]
    bm = 1024

    def body(x_ref, w1_ref, b1_ref, w2_ref, b2_ref, o_ref):
        x = x_ref[...]
        hh = jnp.maximum(
            jnp.dot(x, w1_ref[...], preferred_element_type=jnp.float32)
            + b1_ref[...],
            0.0,
        )
        o_ref[...] = jnp.sum(hh * w2_ref[...], axis=1, keepdims=True) + b2_ref[...]

    return pl.pallas_call(
        body,
        grid=(b_ // bm,),
        in_specs=[
            pl.BlockSpec((bm, k), lambda i: (i, 0)),
            pl.BlockSpec((k, h), lambda i: (0, 0)),
            pl.BlockSpec((1, h), lambda i: (0, 0)),
            pl.BlockSpec((1, h), lambda i: (0, 0)),
            pl.BlockSpec((1, 1), lambda i: (0, 0)),
        ],
        out_specs=pl.BlockSpec((bm, 1), lambda i: (i, 0)),
        out_shape=jax.ShapeDtypeStruct((b_, 1), jnp.float32),
    )(flat, W1, b1r, W2r, b2r)


def kernel(X, table, W1, b1, W2, b2):
    b_, f = X.shape
    v, d = table.shape
    h = W1.shape[1]
    n_rows = b_ * f

    tail = v % _GCHUNK
    tail_pk = table[v - tail :, :].reshape(tail * d // _GCHUNK, _GCHUNK)
    table_pk = _sc_format(table.T, tail_pk, v, d)
    table_lin = table_pk.reshape(v, d)

    idx2d = X.reshape(n_rows // _GCHUNK, _GCHUNK)
    rows = _sc_gather(table_lin, idx2d, n_rows, d)
    flat = rows.reshape(b_, f * d)
    return _tc_mlp(flat, W1, b1.reshape(1, h), W2.reshape(1, h), b2.reshape(1, 1))


# speedup vs baseline: 1.8274x; 1.0630x over previous
import functools

import jax
import jax.numpy as jnp
from jax import lax
from jax.experimental import pallas as pl
from jax.experimental.pallas import tpu as pltpu
from jax.experimental.pallas import tpu_sc as plsc


def _sc_info():
    try:
        info = plsc.get_sparse_core_info()
        return info.num_cores, info.num_subcores
    except Exception:
        return 2, 16


_GCHUNK = 128


def _sc_format(tableT, tail_pk, v, d):
    nc, ns = _sc_info()
    nw = nc * ns
    nblk = v // _GCHUNK
    tail_rows = tail_pk.shape[0]
    out_rows = (v * d) // _GCHUNK

    m = 2
    sb_per_w = nblk // (nw * m)
    nleft = nblk - sb_per_w * nw * m
    scols = m * _GCHUNK
    srows = m * d

    mesh = plsc.VectorSubcoreMesh(core_axis_name="c", subcore_axis_name="s")

    @functools.partial(
        pl.kernel,
        out_type=jax.ShapeDtypeStruct((out_rows, _GCHUNK), jnp.float32),
        mesh=mesh,
        scratch_types=[
            pltpu.VMEM((d, scols), jnp.float32),
            pltpu.VMEM((d, scols), jnp.float32),
            pltpu.VMEM((srows, _GCHUNK), jnp.float32),
            pltpu.VMEM((srows, _GCHUNK), jnp.float32),
            pltpu.VMEM((tail_rows, _GCHUNK), jnp.float32),
            pltpu.SemaphoreType.DMA,
            pltpu.SemaphoreType.DMA,
            pltpu.SemaphoreType.DMA,
            pltpu.SemaphoreType.DMA,
        ],
        compiler_params=pltpu.CompilerParams(
            use_tc_tiling_on_sc=True, needs_layout_passes=False
        ),
    )
    def format_kernel(tableT_hbm, tail_hbm, out_hbm, tin0, tin1, tout0, tout1,
                      ttail, si0, si1, so0, so1):
        wid = lax.axis_index("s") * nc + lax.axis_index("c")
        sb0 = wid * sb_per_w
        d_lo = lax.iota(jnp.int32, 16)
        d_hi = d_lo + 16
        tins = (tin0, tin1)
        touts = (tout0, tout1)
        sis = (si0, si1)
        sos = (so0, so1)

        def src(j):
            return tableT_hbm.at[:, pl.ds((sb0 + j) * scols, scols)]

        def dst(j):
            return out_hbm.at[pl.ds((sb0 + j) * srows, srows)]

        def issue_in(j, p):
            pltpu.async_copy(src(j), tins[p], sis[p])

        def wait_in(j, p):
            pltpu.make_async_copy(src(j), tins[p], sis[p]).wait()

        def issue_out(j, p):
            pltpu.async_copy(touts[p], dst(j), sos[p])

        def wait_out(j, p):
            pltpu.make_async_copy(touts[p], dst(j), sos[p]).wait()

        def compute(tin, tout):
            def rowpair(rr, carry):
                b = rr // 16
                col0 = _GCHUNK * b + 4 * (2 * rr - 32 * b)
                for i in range(2):
                    for k in range(8):
                        dvec = d_lo if (k % 2 == 0) else d_hi
                        cvec = jnp.zeros((16,), jnp.int32) + (
                            col0 + 4 * i + (k // 2)
                        )
                        vals = plsc.load_gather(tin, [dvec, cvec])
                        tout[2 * rr + i, pl.ds(16 * k, 16)] = vals
                return carry

            lax.fori_loop(0, 16 * m, rowpair, 0)

        def steady(j, p, first):
            wait_in(j, p)
            if not first:
                wait_out(j - 2, p)
            compute(tins[p], touts[p])
            issue_out(j, p)
            if j + 2 < sb_per_w:
                issue_in(j + 2, p)

        issue_in(0, 0)
        issue_in(1, 1)
        steady(0, 0, True)
        steady(1, 1, True)

        def body(jj, carry):
            j = 2 + 2 * jj
            wait_in(j, 0)
            wait_out(j - 2, 0)
            compute(tin0, tout0)
            issue_out(j, 0)

            @pl.when(j + 2 < sb_per_w)
            def _():
                pltpu.async_copy(
                    tableT_hbm.at[:, pl.ds((sb0 + j + 2) * scols, scols)],
                    tin0, si0,
                )
            wait_in(j + 1, 1)
            wait_out(j - 1, 1)
            compute(tin1, tout1)
            issue_out(j + 1, 1)

            @pl.when(j + 3 < sb_per_w)
            def _():
                pltpu.async_copy(
                    tableT_hbm.at[:, pl.ds((sb0 + j + 3) * scols, scols)],
                    tin1, si1,
                )

            return carry

        if sb_per_w > 2:
            nst = (sb_per_w - 2) // 2
            lax.fori_loop(0, nst, body, 0)
            for j in range(2 + 2 * nst, sb_per_w):
                p = j % 2
                wait_in(j, p)
                wait_out(j - 2, p)
                compute(tins[p], touts[p])
                issue_out(j, p)
        wait_out(sb_per_w - 2, sb_per_w % 2)
        wait_out(sb_per_w - 1, (sb_per_w - 1) % 2)

        if nleft:
            @pl.when(wid < nleft)
            def _():
                g = nblk - nleft + wid
                pltpu.sync_copy(
                    tableT_hbm.at[:, pl.ds(g * _GCHUNK, _GCHUNK)],
                    tin0.at[:, pl.ds(0, _GCHUNK)],
                )
                def rowpair(rr, carry):
                    for i in range(2):
                        for k in range(8):
                            dvec = d_lo if (k % 2 == 0) else d_hi
                            cvec = jnp.zeros((16,), jnp.int32) + (
                                4 * (2 * rr + i) + (k // 2)
                            )
                            vals = plsc.load_gather(tin0, [dvec, cvec])
                            tout0[2 * rr + i, pl.ds(16 * k, 16)] = vals
                    return carry

                lax.fori_loop(0, 16, rowpair, 0)
                pltpu.sync_copy(
                    tout0.at[pl.ds(0, 32)], out_hbm.at[pl.ds(g * 32, 32)]
                )

        @pl.when(wid == 0)
        def _():
            pltpu.sync_copy(tail_hbm, ttail)
            pltpu.sync_copy(ttail, out_hbm.at[pl.ds(nblk * 32, tail_rows)])

    return format_kernel(tableT, tail_pk)


def _sc_gather(table_lin, idx2d, n_rows, d):
    nc, ns = _sc_info()
    nw = nc * ns
    chunks_total = n_rows // _GCHUNK
    chunks_per_w = chunks_total // nw
    rows_per_w = chunks_per_w * _GCHUNK

    mesh = plsc.VectorSubcoreMesh(core_axis_name="c", subcore_axis_name="s")

    nq = chunks_per_w // 4

    @functools.partial(
        pl.kernel,
        out_type=jax.ShapeDtypeStruct((n_rows, d), jnp.float32),
        mesh=mesh,
        scratch_types=[
            pltpu.VMEM((chunks_per_w, _GCHUNK), jnp.int32),
            pltpu.VMEM((_GCHUNK, d), jnp.float32),
            pltpu.VMEM((_GCHUNK, d), jnp.float32),
            pltpu.VMEM((_GCHUNK, d), jnp.float32),
            pltpu.VMEM((_GCHUNK, d), jnp.float32),
            pltpu.SemaphoreType.DMA,
            pltpu.SemaphoreType.DMA,
            pltpu.SemaphoreType.DMA,
            pltpu.SemaphoreType.DMA,
            pltpu.SemaphoreType.DMA,
            pltpu.SemaphoreType.DMA,
            pltpu.SemaphoreType.DMA,
            pltpu.SemaphoreType.DMA,
        ],
        compiler_params=pltpu.CompilerParams(use_tc_tiling_on_sc=False),
    )
    def gather_kernel(table_hbm, idx_hbm, out_hbm, idx_v, r0, r1, r2, r3,
                      g0, g1, g2, g3, o0, o1, o2, o3):
        wid = lax.axis_index("s") * nc + lax.axis_index("c")
        cbase = wid * chunks_per_w
        rbase = wid * rows_per_w
        pltpu.sync_copy(idx_hbm.at[pl.ds(cbase, chunks_per_w)], idx_v)
        rows = (r0, r1, r2, r3)
        sgs = (g0, g1, g2, g3)
        sos = (o0, o1, o2, o3)

        def gsrc(j):
            return table_hbm.at[idx_v.at[j]]

        def odst(j):
            return out_hbm.at[pl.ds(rbase + j * _GCHUNK, _GCHUNK)]

        def issue_g(j, p):
            pltpu.async_copy(gsrc(j), rows[p], sgs[p])

        def wait_g(j, p):
            pltpu.make_async_copy(gsrc(j), rows[p], sgs[p]).wait()

        def issue_o(j, p):
            pltpu.async_copy(rows[p], odst(j), sos[p])

        def wait_o(j, p):
            pltpu.make_async_copy(rows[p], odst(j), sos[p]).wait()

        issue_g(0, 0)
        issue_g(1, 1)
        wait_g(0, 0)
        issue_o(0, 0)
        issue_g(2, 2)
        wait_g(1, 1)
        issue_o(1, 1)
        issue_g(3, 3)
        wait_g(2, 2)
        issue_o(2, 2)
        wait_o(0, 0)
        issue_g(4, 0)
        wait_g(3, 3)
        issue_o(3, 3)
        wait_o(1, 1)
        issue_g(5, 1)

        def body(jj, carry):
            for q in range(4):
                s = 4 * jj + q
                p = q
                wait_g(s, p)
                issue_o(s, p)
                wait_o(s - 2, (q + 2) % 4)

                @pl.when(s + 2 < chunks_per_w)
                def _():
                    issue_g(s + 2, (q + 2) % 4)

            return carry

        lax.fori_loop(1, nq, body, 0)
        wait_o(chunks_per_w - 2, 2)
        wait_o(chunks_per_w - 1, 3)

    return gather_kernel(table_lin, idx2d)


def _tc_mlp(flat, W1, b1r, W2r, b2r):
    b_, k = flat.shape
    h = W1.shape[1]
    bm = 1024

    def body(x_ref, w1_ref, b1_ref, w2_ref, b2_ref, o_ref):
        x = x_ref[...]
        hh = jnp.maximum(
            jnp.dot(x, w1_ref[...], preferred_element_type=jnp.float32)
            + b1_ref[...],
            0.0,
        )
        o_ref[...] = jnp.sum(hh * w2_ref[...], axis=1, keepdims=True) + b2_ref[...]

    return pl.pallas_call(
        body,
        grid=(b_ // bm,),
        in_specs=[
            pl.BlockSpec((bm, k), lambda i: (i, 0)),
            pl.BlockSpec((k, h), lambda i: (0, 0)),
            pl.BlockSpec((1, h), lambda i: (0, 0)),
            pl.BlockSpec((1, h), lambda i: (0, 0)),
            pl.BlockSpec((1, 1), lambda i: (0, 0)),
        ],
        out_specs=pl.BlockSpec((bm, 1), lambda i: (i, 0)),
        out_shape=jax.ShapeDtypeStruct((b_, 1), jnp.float32),
    )(flat, W1, b1r, W2r, b2r)


def kernel(X, table, W1, b1, W2, b2):
    b_, f = X.shape
    v, d = table.shape
    h = W1.shape[1]
    n_rows = b_ * f

    tail = v % _GCHUNK
    tail_pk = table[v - tail :, :].reshape(tail * d // _GCHUNK, _GCHUNK)
    table_pk = _sc_format(table.T, tail_pk, v, d)
    table_lin = table_pk.reshape(v, d)

    idx2d = X.reshape(n_rows // _GCHUNK, _GCHUNK)
    rows = _sc_gather(table_lin, idx2d, n_rows, d)
    flat = rows.reshape(b_, f * d)
    return _tc_mlp(flat, W1, b1.reshape(1, h), W2.reshape(1, h), b2.reshape(1, 1))


# speedup vs baseline: 2.4735x; 1.3536x over previous
import functools

import jax
import jax.numpy as jnp
from jax import lax
from jax.experimental import pallas as pl
from jax.experimental.pallas import tpu as pltpu
from jax.experimental.pallas import tpu_sc as plsc


def _sc_info():
    try:
        info = plsc.get_sparse_core_info()
        return info.num_cores, info.num_subcores
    except Exception:
        return 2, 16


_GCHUNK = 128


def _sc_format(tableT, tail_pk, v, d):
    nc, ns = _sc_info()
    nw = nc * ns
    nblk = v // _GCHUNK
    tail_rows = tail_pk.shape[0]
    out_rows = (v * d) // _GCHUNK

    m = 2
    sb_per_w = nblk // (nw * m)
    nleft = nblk - sb_per_w * nw * m
    scols = m * _GCHUNK
    srows = m * d

    mesh = plsc.VectorSubcoreMesh(core_axis_name="c", subcore_axis_name="s")

    @functools.partial(
        pl.kernel,
        out_type=jax.ShapeDtypeStruct((out_rows, _GCHUNK), jnp.float32),
        mesh=mesh,
        scratch_types=[
            pltpu.VMEM((d, scols), jnp.float32),
            pltpu.VMEM((d, scols), jnp.float32),
            pltpu.VMEM((srows, _GCHUNK), jnp.float32),
            pltpu.VMEM((srows, _GCHUNK), jnp.float32),
            pltpu.VMEM((d, scols), jnp.float32),
            pltpu.VMEM((tail_rows, _GCHUNK), jnp.float32),
            pltpu.SemaphoreType.DMA,
            pltpu.SemaphoreType.DMA,
            pltpu.SemaphoreType.DMA,
            pltpu.SemaphoreType.DMA,
        ],
        compiler_params=pltpu.CompilerParams(
            use_tc_tiling_on_sc=True, needs_layout_passes=False
        ),
    )
    def format_kernel(tableT_hbm, tail_hbm, out_hbm, tin0, tin1, tout0, tout1,
                      tskew, ttail, si0, si1, so0, so1):
        wid = lax.axis_index("s") * nc + lax.axis_index("c")
        sb0 = wid * sb_per_w
        d_lo = lax.iota(jnp.int32, 16)
        d_hi = d_lo + 16
        tins = (tin0, tin1)
        touts = (tout0, tout1)
        sis = (si0, si1)
        sos = (so0, so1)

        def src(j):
            return tableT_hbm.at[:, pl.ds((sb0 + j) * scols, scols)]

        def dst(j):
            return out_hbm.at[pl.ds((sb0 + j) * srows, srows)]

        def issue_in(j, p):
            pltpu.async_copy(src(j), tins[p], sis[p])

        def wait_in(j, p):
            pltpu.make_async_copy(src(j), tins[p], sis[p]).wait()

        def issue_out(j, p):
            pltpu.async_copy(touts[p], dst(j), sos[p])

        def wait_out(j, p):
            pltpu.make_async_copy(touts[p], dst(j), sos[p]).wait()

        def compute(tin, tout):
            def skew_row(dd, carry):
                rot = (d_lo + dd) & 15
                for grp in range(scols // 16):
                    idxa = rot + (16 * grp)
                    vals = plsc.load_gather(tin, [jnp.zeros((16,), jnp.int32) + dd, idxa])
                    tskew[dd, pl.ds(16 * grp, 16)] = vals
                return carry

            lax.fori_loop(0, d, skew_row, 0)

            def outrow(u, carry):
                for q in range(4):
                    c = 4 * u + q
                    cg0 = (c >> 4) << 4
                    idxb = (((jnp.zeros((16,), jnp.int32) + (c - cg0 + 32)) - d_lo) & 15) + cg0
                    for h in range(2):
                        dvec = d_lo if h == 0 else d_hi
                        vals = plsc.load_gather(tskew, [dvec, idxb])
                        tout[u, pl.ds(q * 32 + 16 * h, 16)] = vals
                return carry

            lax.fori_loop(0, srows, outrow, 0)

        def steady(j, p, first):
            wait_in(j, p)
            if not first:
                wait_out(j - 2, p)
            compute(tins[p], touts[p])
            issue_out(j, p)
            if j + 2 < sb_per_w:
                issue_in(j + 2, p)

        issue_in(0, 0)
        issue_in(1, 1)
        steady(0, 0, True)
        steady(1, 1, True)

        def body(jj, carry):
            j = 2 + 2 * jj
            wait_in(j, 0)
            wait_out(j - 2, 0)
            compute(tin0, tout0)
            issue_out(j, 0)

            @pl.when(j + 2 < sb_per_w)
            def _():
                pltpu.async_copy(
                    tableT_hbm.at[:, pl.ds((sb0 + j + 2) * scols, scols)],
                    tin0, si0,
                )
            wait_in(j + 1, 1)
            wait_out(j - 1, 1)
            compute(tin1, tout1)
            issue_out(j + 1, 1)

            @pl.when(j + 3 < sb_per_w)
            def _():
                pltpu.async_copy(
                    tableT_hbm.at[:, pl.ds((sb0 + j + 3) * scols, scols)],
                    tin1, si1,
                )

            return carry

        if sb_per_w > 2:
            nst = (sb_per_w - 2) // 2
            lax.fori_loop(0, nst, body, 0)
            for j in range(2 + 2 * nst, sb_per_w):
                p = j % 2
                wait_in(j, p)
                wait_out(j - 2, p)
                compute(tins[p], touts[p])
                issue_out(j, p)
        wait_out(sb_per_w - 2, sb_per_w % 2)
        wait_out(sb_per_w - 1, (sb_per_w - 1) % 2)

        if nleft:
            @pl.when(wid < nleft)
            def _():
                g = nblk - nleft + wid
                pltpu.sync_copy(
                    tableT_hbm.at[:, pl.ds(g * _GCHUNK, _GCHUNK)],
                    tin0.at[:, pl.ds(0, _GCHUNK)],
                )
                def rowpair(rr, carry):
                    for i in range(2):
                        for k in range(8):
                            dvec = d_lo if (k % 2 == 0) else d_hi
                            cvec = jnp.zeros((16,), jnp.int32) + (
                                4 * (2 * rr + i) + (k // 2)
                            )
                            vals = plsc.load_gather(tin0, [dvec, cvec])
                            tout0[2 * rr + i, pl.ds(16 * k, 16)] = vals
                    return carry

                lax.fori_loop(0, 16, rowpair, 0)
                pltpu.sync_copy(
                    tout0.at[pl.ds(0, 32)], out_hbm.at[pl.ds(g * 32, 32)]
                )

        @pl.when(wid == 0)
        def _():
            pltpu.sync_copy(tail_hbm, ttail)
            pltpu.sync_copy(ttail, out_hbm.at[pl.ds(nblk * 32, tail_rows)])

    return format_kernel(tableT, tail_pk)


def _sc_gather(table_lin, idx2d, n_rows, d):
    nc, ns = _sc_info()
    nw = nc * ns
    chunks_total = n_rows // _GCHUNK
    chunks_per_w = chunks_total // nw
    rows_per_w = chunks_per_w * _GCHUNK

    mesh = plsc.VectorSubcoreMesh(core_axis_name="c", subcore_axis_name="s")

    nq = chunks_per_w // 4

    @functools.partial(
        pl.kernel,
        out_type=jax.ShapeDtypeStruct((n_rows, d), jnp.float32),
        mesh=mesh,
        scratch_types=[
            pltpu.VMEM((chunks_per_w, _GCHUNK), jnp.int32),
            pltpu.VMEM((_GCHUNK, d), jnp.float32),
            pltpu.VMEM((_GCHUNK, d), jnp.float32),
            pltpu.VMEM((_GCHUNK, d), jnp.float32),
            pltpu.VMEM((_GCHUNK, d), jnp.float32),
            pltpu.SemaphoreType.DMA,
            pltpu.SemaphoreType.DMA,
            pltpu.SemaphoreType.DMA,
            pltpu.SemaphoreType.DMA,
            pltpu.SemaphoreType.DMA,
            pltpu.SemaphoreType.DMA,
            pltpu.SemaphoreType.DMA,
            pltpu.SemaphoreType.DMA,
        ],
        compiler_params=pltpu.CompilerParams(use_tc_tiling_on_sc=False),
    )
    def gather_kernel(table_hbm, idx_hbm, out_hbm, idx_v, r0, r1, r2, r3,
                      g0, g1, g2, g3, o0, o1, o2, o3):
        wid = lax.axis_index("s") * nc + lax.axis_index("c")
        cbase = wid * chunks_per_w
        rbase = wid * rows_per_w
        pltpu.sync_copy(idx_hbm.at[pl.ds(cbase, chunks_per_w)], idx_v)
        rows = (r0, r1, r2, r3)
        sgs = (g0, g1, g2, g3)
        sos = (o0, o1, o2, o3)

        def gsrc(j):
            return table_hbm.at[idx_v.at[j]]

        def odst(j):
            return out_hbm.at[pl.ds(rbase + j * _GCHUNK, _GCHUNK)]

        def issue_g(j, p):
            pltpu.async_copy(gsrc(j), rows[p], sgs[p])

        def wait_g(j, p):
            pltpu.make_async_copy(gsrc(j), rows[p], sgs[p]).wait()

        def issue_o(j, p):
            pltpu.async_copy(rows[p], odst(j), sos[p])

        def wait_o(j, p):
            pltpu.make_async_copy(rows[p], odst(j), sos[p]).wait()

        issue_g(0, 0)
        issue_g(1, 1)
        wait_g(0, 0)
        issue_o(0, 0)
        issue_g(2, 2)
        wait_g(1, 1)
        issue_o(1, 1)
        issue_g(3, 3)
        wait_g(2, 2)
        issue_o(2, 2)
        wait_o(0, 0)
        issue_g(4, 0)
        wait_g(3, 3)
        issue_o(3, 3)
        wait_o(1, 1)
        issue_g(5, 1)

        def body(jj, carry):
            for q in range(4):
                s = 4 * jj + q
                p = q
                wait_g(s, p)
                issue_o(s, p)
                wait_o(s - 2, (q + 2) % 4)

                @pl.when(s + 2 < chunks_per_w)
                def _():
                    issue_g(s + 2, (q + 2) % 4)

            return carry

        lax.fori_loop(1, nq, body, 0)
        wait_o(chunks_per_w - 2, 2)
        wait_o(chunks_per_w - 1, 3)

    return gather_kernel(table_lin, idx2d)


def _tc_mlp(flat, W1, b1r, W2r, b2r):
    b_, k = flat.shape
    h = W1.shape[1]
    bm = 1024

    def body(x_ref, w1_ref, b1_ref, w2_ref, b2_ref, o_ref):
        x = x_ref[...]
        hh = jnp.maximum(
            jnp.dot(x, w1_ref[...], preferred_element_type=jnp.float32)
            + b1_ref[...],
            0.0,
        )
        o_ref[...] = jnp.sum(hh * w2_ref[...], axis=1, keepdims=True) + b2_ref[...]

    return pl.pallas_call(
        body,
        grid=(b_ // bm,),
        in_specs=[
            pl.BlockSpec((bm, k), lambda i: (i, 0)),
            pl.BlockSpec((k, h), lambda i: (0, 0)),
            pl.BlockSpec((1, h), lambda i: (0, 0)),
            pl.BlockSpec((1, h), lambda i: (0, 0)),
            pl.BlockSpec((1, 1), lambda i: (0, 0)),
        ],
        out_specs=pl.BlockSpec((bm, 1), lambda i: (i, 0)),
        out_shape=jax.ShapeDtypeStruct((b_, 1), jnp.float32),
    )(flat, W1, b1r, W2r, b2r)


def kernel(X, table, W1, b1, W2, b2):
    b_, f = X.shape
    v, d = table.shape
    h = W1.shape[1]
    n_rows = b_ * f

    tail = v % _GCHUNK
    tail_pk = table[v - tail :, :].reshape(tail * d // _GCHUNK, _GCHUNK)
    table_pk = _sc_format(table.T, tail_pk, v, d)
    table_lin = table_pk.reshape(v, d)

    idx2d = X.reshape(n_rows // _GCHUNK, _GCHUNK)
    rows = _sc_gather(table_lin, idx2d, n_rows, d)
    flat = rows.reshape(b_, f * d)
    return _tc_mlp(flat, W1, b1.reshape(1, h), W2.reshape(1, h), b2.reshape(1, 1))


# speedup vs baseline: 2.4945x; 1.0085x over previous
import functools

import jax
import jax.numpy as jnp
from jax import lax
from jax.experimental import pallas as pl
from jax.experimental.pallas import tpu as pltpu
from jax.experimental.pallas import tpu_sc as plsc


def _sc_info():
    try:
        info = plsc.get_sparse_core_info()
        return info.num_cores, info.num_subcores
    except Exception:
        return 2, 16


_GCHUNK = 128


def _sc_format(tableT, tail_pk, v, d):
    nc, ns = _sc_info()
    nw = nc * ns
    nblk = v // _GCHUNK
    tail_rows = tail_pk.shape[0]
    out_rows = (v * d) // _GCHUNK

    m = 2
    sb_per_w = nblk // (nw * m)
    nleft = nblk - sb_per_w * nw * m
    scols = m * _GCHUNK
    srows = m * d

    mesh = plsc.VectorSubcoreMesh(core_axis_name="c", subcore_axis_name="s")

    @functools.partial(
        pl.kernel,
        out_type=jax.ShapeDtypeStruct((out_rows, _GCHUNK), jnp.float32),
        mesh=mesh,
        scratch_types=[
            pltpu.VMEM((d, scols), jnp.float32),
            pltpu.VMEM((d, scols), jnp.float32),
            pltpu.VMEM((srows, _GCHUNK), jnp.float32),
            pltpu.VMEM((srows, _GCHUNK), jnp.float32),
            pltpu.VMEM((d, scols), jnp.float32),
            pltpu.VMEM((tail_rows, _GCHUNK), jnp.float32),
            pltpu.SemaphoreType.DMA,
            pltpu.SemaphoreType.DMA,
            pltpu.SemaphoreType.DMA,
            pltpu.SemaphoreType.DMA,
        ],
        compiler_params=pltpu.CompilerParams(
            use_tc_tiling_on_sc=True, needs_layout_passes=False
        ),
    )
    def format_kernel(tableT_hbm, tail_hbm, out_hbm, tin0, tin1, tout0, tout1,
                      tskew, ttail, si0, si1, so0, so1):
        wid = lax.axis_index("s") * nc + lax.axis_index("c")
        sb0 = wid * sb_per_w
        d_lo = lax.iota(jnp.int32, 16)
        d_hi = d_lo + 16
        tins = (tin0, tin1)
        touts = (tout0, tout1)
        sis = (si0, si1)
        sos = (so0, so1)

        def src(j):
            return tableT_hbm.at[:, pl.ds((sb0 + j) * scols, scols)]

        def dst(j):
            return out_hbm.at[pl.ds((sb0 + j) * srows, srows)]

        def issue_in(j, p):
            pltpu.async_copy(src(j), tins[p], sis[p])

        def wait_in(j, p):
            pltpu.make_async_copy(src(j), tins[p], sis[p]).wait()

        def issue_out(j, p):
            pltpu.async_copy(touts[p], dst(j), sos[p])

        def wait_out(j, p):
            pltpu.make_async_copy(touts[p], dst(j), sos[p]).wait()

        def compute(tin, tout):
            def skew_row(dp, carry):
                for i in range(2):
                    dd = 2 * dp + i
                    rot = (d_lo + dd) & 15
                    dsplat = jnp.zeros((16,), jnp.int32) + dd
                    for grp in range(scols // 16):
                        idxa = rot + (16 * grp)
                        vals = plsc.load_gather(tin, [dsplat, idxa])
                        tskew[dd, pl.ds(16 * grp, 16)] = vals
                return carry

            lax.fori_loop(0, d // 2, skew_row, 0)

            def outrow(up, carry):
                for i in range(2):
                    u = 2 * up + i
                    c0 = 4 * u
                    cg0 = (c0 >> 4) << 4
                    base = (jnp.zeros((16,), jnp.int32) + (c0 - cg0 + 32)) - d_lo
                    for q in range(4):
                        idxb = ((base + q) & 15) + cg0
                        for h in range(2):
                            dvec = d_lo if h == 0 else d_hi
                            vals = plsc.load_gather(tskew, [dvec, idxb])
                            tout[u, pl.ds(q * 32 + 16 * h, 16)] = vals
                return carry

            lax.fori_loop(0, srows // 2, outrow, 0)

        def steady(j, p, first):
            wait_in(j, p)
            if not first:
                wait_out(j - 2, p)
            compute(tins[p], touts[p])
            issue_out(j, p)
            if j + 2 < sb_per_w:
                issue_in(j + 2, p)

        issue_in(0, 0)
        issue_in(1, 1)
        steady(0, 0, True)
        steady(1, 1, True)

        def body(jj, carry):
            j = 2 + 2 * jj
            wait_in(j, 0)
            wait_out(j - 2, 0)
            compute(tin0, tout0)
            issue_out(j, 0)

            @pl.when(j + 2 < sb_per_w)
            def _():
                pltpu.async_copy(
                    tableT_hbm.at[:, pl.ds((sb0 + j + 2) * scols, scols)],
                    tin0, si0,
                )
            wait_in(j + 1, 1)
            wait_out(j - 1, 1)
            compute(tin1, tout1)
            issue_out(j + 1, 1)

            @pl.when(j + 3 < sb_per_w)
            def _():
                pltpu.async_copy(
                    tableT_hbm.at[:, pl.ds((sb0 + j + 3) * scols, scols)],
                    tin1, si1,
                )

            return carry

        if sb_per_w > 2:
            nst = (sb_per_w - 2) // 2
            lax.fori_loop(0, nst, body, 0)
            for j in range(2 + 2 * nst, sb_per_w):
                p = j % 2
                wait_in(j, p)
                wait_out(j - 2, p)
                compute(tins[p], touts[p])
                issue_out(j, p)
        wait_out(sb_per_w - 2, sb_per_w % 2)
        wait_out(sb_per_w - 1, (sb_per_w - 1) % 2)

        if nleft:
            @pl.when(wid < nleft)
            def _():
                g = nblk - nleft + wid
                pltpu.sync_copy(
                    tableT_hbm.at[:, pl.ds(g * _GCHUNK, _GCHUNK)],
                    tin0.at[:, pl.ds(0, _GCHUNK)],
                )
                def rowpair(rr, carry):
                    for i in range(2):
                        for k in range(8):
                            dvec = d_lo if (k % 2 == 0) else d_hi
                            cvec = jnp.zeros((16,), jnp.int32) + (
                                4 * (2 * rr + i) + (k // 2)
                            )
                            vals = plsc.load_gather(tin0, [dvec, cvec])
                            tout0[2 * rr + i, pl.ds(16 * k, 16)] = vals
                    return carry

                lax.fori_loop(0, 16, rowpair, 0)
                pltpu.sync_copy(
                    tout0.at[pl.ds(0, 32)], out_hbm.at[pl.ds(g * 32, 32)]
                )

        @pl.when(wid == 0)
        def _():
            pltpu.sync_copy(tail_hbm, ttail)
            pltpu.sync_copy(ttail, out_hbm.at[pl.ds(nblk * 32, tail_rows)])

    return format_kernel(tableT, tail_pk)


def _sc_gather(table_lin, idx2d, n_rows, d):
    nc, ns = _sc_info()
    nw = nc * ns
    chunks_total = n_rows // _GCHUNK
    chunks_per_w = chunks_total // nw
    rows_per_w = chunks_per_w * _GCHUNK

    mesh = plsc.VectorSubcoreMesh(core_axis_name="c", subcore_axis_name="s")

    nq = chunks_per_w // 4

    @functools.partial(
        pl.kernel,
        out_type=jax.ShapeDtypeStruct((n_rows, d), jnp.float32),
        mesh=mesh,
        scratch_types=[
            pltpu.VMEM((chunks_per_w, _GCHUNK), jnp.int32),
            pltpu.VMEM((_GCHUNK, d), jnp.float32),
            pltpu.VMEM((_GCHUNK, d), jnp.float32),
            pltpu.VMEM((_GCHUNK, d), jnp.float32),
            pltpu.VMEM((_GCHUNK, d), jnp.float32),
            pltpu.SemaphoreType.DMA,
            pltpu.SemaphoreType.DMA,
            pltpu.SemaphoreType.DMA,
            pltpu.SemaphoreType.DMA,
            pltpu.SemaphoreType.DMA,
            pltpu.SemaphoreType.DMA,
            pltpu.SemaphoreType.DMA,
            pltpu.SemaphoreType.DMA,
        ],
        compiler_params=pltpu.CompilerParams(use_tc_tiling_on_sc=False),
    )
    def gather_kernel(table_hbm, idx_hbm, out_hbm, idx_v, r0, r1, r2, r3,
                      g0, g1, g2, g3, o0, o1, o2, o3):
        wid = lax.axis_index("s") * nc + lax.axis_index("c")
        cbase = wid * chunks_per_w
        rbase = wid * rows_per_w
        pltpu.sync_copy(idx_hbm.at[pl.ds(cbase, chunks_per_w)], idx_v)
        rows = (r0, r1, r2, r3)
        sgs = (g0, g1, g2, g3)
        sos = (o0, o1, o2, o3)

        def gsrc(j):
            return table_hbm.at[idx_v.at[j]]

        def odst(j):
            return out_hbm.at[pl.ds(rbase + j * _GCHUNK, _GCHUNK)]

        def issue_g(j, p):
            pltpu.async_copy(gsrc(j), rows[p], sgs[p])

        def wait_g(j, p):
            pltpu.make_async_copy(gsrc(j), rows[p], sgs[p]).wait()

        def issue_o(j, p):
            pltpu.async_copy(rows[p], odst(j), sos[p])

        def wait_o(j, p):
            pltpu.make_async_copy(rows[p], odst(j), sos[p]).wait()

        issue_g(0, 0)
        issue_g(1, 1)
        wait_g(0, 0)
        issue_o(0, 0)
        issue_g(2, 2)
        wait_g(1, 1)
        issue_o(1, 1)
        issue_g(3, 3)
        wait_g(2, 2)
        issue_o(2, 2)
        wait_o(0, 0)
        issue_g(4, 0)
        wait_g(3, 3)
        issue_o(3, 3)
        wait_o(1, 1)
        issue_g(5, 1)

        def body(jj, carry):
            for q in range(4):
                s = 4 * jj + q
                p = q
                wait_g(s, p)
                issue_o(s, p)
                wait_o(s - 2, (q + 2) % 4)

                @pl.when(s + 2 < chunks_per_w)
                def _():
                    issue_g(s + 2, (q + 2) % 4)

            return carry

        lax.fori_loop(1, nq, body, 0)
        wait_o(chunks_per_w - 2, 2)
        wait_o(chunks_per_w - 1, 3)

    return gather_kernel(table_lin, idx2d)


def _tc_mlp(flat, W1, b1r, W2r, b2r):
    b_, k = flat.shape
    h = W1.shape[1]
    bm = 1024

    def body(x_ref, w1_ref, b1_ref, w2_ref, b2_ref, o_ref):
        x = x_ref[...]
        hh = jnp.maximum(
            jnp.dot(x, w1_ref[...], preferred_element_type=jnp.float32)
            + b1_ref[...],
            0.0,
        )
        o_ref[...] = jnp.sum(hh * w2_ref[...], axis=1, keepdims=True) + b2_ref[...]

    return pl.pallas_call(
        body,
        grid=(b_ // bm,),
        in_specs=[
            pl.BlockSpec((bm, k), lambda i: (i, 0)),
            pl.BlockSpec((k, h), lambda i: (0, 0)),
            pl.BlockSpec((1, h), lambda i: (0, 0)),
            pl.BlockSpec((1, h), lambda i: (0, 0)),
            pl.BlockSpec((1, 1), lambda i: (0, 0)),
        ],
        out_specs=pl.BlockSpec((bm, 1), lambda i: (i, 0)),
        out_shape=jax.ShapeDtypeStruct((b_, 1), jnp.float32),
    )(flat, W1, b1r, W2r, b2r)


def kernel(X, table, W1, b1, W2, b2):
    b_, f = X.shape
    v, d = table.shape
    h = W1.shape[1]
    n_rows = b_ * f

    tail = v % _GCHUNK
    tail_pk = table[v - tail :, :].reshape(tail * d // _GCHUNK, _GCHUNK)
    table_pk = _sc_format(table.T, tail_pk, v, d)
    table_lin = table_pk.reshape(v, d)

    idx2d = X.reshape(n_rows // _GCHUNK, _GCHUNK)
    rows = _sc_gather(table_lin, idx2d, n_rows, d)
    flat = rows.reshape(b_, f * d)
    return _tc_mlp(flat, W1, b1.reshape(1, h), W2.reshape(1, h), b2.reshape(1, 1))


# speedup vs baseline: 5.5355x; 2.2191x over previous
import functools

import jax
import jax.numpy as jnp
from jax import lax
from jax.experimental import pallas as pl
from jax.experimental.pallas import tpu as pltpu
from jax.experimental.pallas import tpu_sc as plsc


def _sc_info():
    try:
        info = plsc.get_sparse_core_info()
        return info.num_cores, info.num_subcores
    except Exception:
        return 2, 16


_GCHUNK = 128


def _sc_format(tableT, tail_pk, v, d):
    nc, ns = _sc_info()
    nw = nc * ns
    nblk = v // _GCHUNK
    tail_rows = tail_pk.shape[0]
    out_rows = (v * d) // _GCHUNK

    m = 2
    sb_per_w = nblk // (nw * m)
    nleft = nblk - sb_per_w * nw * m
    scols = m * _GCHUNK
    srows = m * d

    mesh = plsc.VectorSubcoreMesh(core_axis_name="c", subcore_axis_name="s")

    @functools.partial(
        pl.kernel,
        out_type=jax.ShapeDtypeStruct((out_rows, _GCHUNK), jnp.float32),
        mesh=mesh,
        scratch_types=[
            pltpu.VMEM((d, scols), jnp.float32),
            pltpu.VMEM((d, scols), jnp.float32),
            pltpu.VMEM((srows, _GCHUNK), jnp.float32),
            pltpu.VMEM((srows, _GCHUNK), jnp.float32),
            pltpu.VMEM((d, scols), jnp.float32),
            pltpu.VMEM((tail_rows, _GCHUNK), jnp.float32),
            pltpu.SemaphoreType.DMA,
            pltpu.SemaphoreType.DMA,
            pltpu.SemaphoreType.DMA,
            pltpu.SemaphoreType.DMA,
        ],
        compiler_params=pltpu.CompilerParams(
            use_tc_tiling_on_sc=True, needs_layout_passes=False
        ),
    )
    def format_kernel(tableT_hbm, tail_hbm, out_hbm, tin0, tin1, tout0, tout1,
                      tskew, ttail, si0, si1, so0, so1):
        wid = lax.axis_index("s") * nc + lax.axis_index("c")
        sb0 = wid * sb_per_w
        d_lo = lax.iota(jnp.int32, 16)
        d_hi = d_lo + 16
        tins = (tin0, tin1)
        touts = (tout0, tout1)
        sis = (si0, si1)
        sos = (so0, so1)

        def src(j):
            return tableT_hbm.at[:, pl.ds((sb0 + j) * scols, scols)]

        def dst(j):
            return out_hbm.at[pl.ds((sb0 + j) * srows, srows)]

        def issue_in(j, p):
            pltpu.async_copy(src(j), tins[p], sis[p])

        def wait_in(j, p):
            pltpu.make_async_copy(src(j), tins[p], sis[p]).wait()

        def issue_out(j, p):
            pltpu.async_copy(touts[p], dst(j), sos[p])

        def wait_out(j, p):
            pltpu.make_async_copy(touts[p], dst(j), sos[p]).wait()

        def compute(tin, tout):
            @plsc.parallel_loop(0, d, unroll=2)
            def _(dd):
                rot = (d_lo + dd) & 15
                dsplat = jnp.zeros((16,), jnp.int32) + dd
                for grp in range(scols // 16):
                    idxa = rot + (16 * grp)
                    vals = plsc.load_gather(tin, [dsplat, idxa])
                    tskew[dd, pl.ds(16 * grp, 16)] = vals

            @plsc.parallel_loop(0, srows, unroll=2)
            def _(u):
                c0 = 4 * u
                cg0 = (c0 >> 4) << 4
                base = (jnp.zeros((16,), jnp.int32) + (c0 - cg0 + 32)) - d_lo
                for q in range(4):
                    idxb = ((base + q) & 15) + cg0
                    for h in range(2):
                        dvec = d_lo if h == 0 else d_hi
                        vals = plsc.load_gather(tskew, [dvec, idxb])
                        tout[u, pl.ds(q * 32 + 16 * h, 16)] = vals

        def steady(j, p, first):
            wait_in(j, p)
            if not first:
                wait_out(j - 2, p)
            compute(tins[p], touts[p])
            issue_out(j, p)
            if j + 2 < sb_per_w:
                issue_in(j + 2, p)

        issue_in(0, 0)
        issue_in(1, 1)
        steady(0, 0, True)
        steady(1, 1, True)

        def body(jj, carry):
            j = 2 + 2 * jj
            wait_in(j, 0)
            wait_out(j - 2, 0)
            compute(tin0, tout0)
            issue_out(j, 0)

            @pl.when(j + 2 < sb_per_w)
            def _():
                pltpu.async_copy(
                    tableT_hbm.at[:, pl.ds((sb0 + j + 2) * scols, scols)],
                    tin0, si0,
                )
            wait_in(j + 1, 1)
            wait_out(j - 1, 1)
            compute(tin1, tout1)
            issue_out(j + 1, 1)

            @pl.when(j + 3 < sb_per_w)
            def _():
                pltpu.async_copy(
                    tableT_hbm.at[:, pl.ds((sb0 + j + 3) * scols, scols)],
                    tin1, si1,
                )

            return carry

        if sb_per_w > 2:
            nst = (sb_per_w - 2) // 2
            lax.fori_loop(0, nst, body, 0)
            for j in range(2 + 2 * nst, sb_per_w):
                p = j % 2
                wait_in(j, p)
                wait_out(j - 2, p)
                compute(tins[p], touts[p])
                issue_out(j, p)
        wait_out(sb_per_w - 2, sb_per_w % 2)
        wait_out(sb_per_w - 1, (sb_per_w - 1) % 2)

        if nleft:
            @pl.when(wid < nleft)
            def _():
                g = nblk - nleft + wid
                pltpu.sync_copy(
                    tableT_hbm.at[:, pl.ds(g * _GCHUNK, _GCHUNK)],
                    tin0.at[:, pl.ds(0, _GCHUNK)],
                )
                def rowpair(rr, carry):
                    for i in range(2):
                        for k in range(8):
                            dvec = d_lo if (k % 2 == 0) else d_hi
                            cvec = jnp.zeros((16,), jnp.int32) + (
                                4 * (2 * rr + i) + (k // 2)
                            )
                            vals = plsc.load_gather(tin0, [dvec, cvec])
                            tout0[2 * rr + i, pl.ds(16 * k, 16)] = vals
                    return carry

                lax.fori_loop(0, 16, rowpair, 0)
                pltpu.sync_copy(
                    tout0.at[pl.ds(0, 32)], out_hbm.at[pl.ds(g * 32, 32)]
                )

        @pl.when(wid == 0)
        def _():
            pltpu.sync_copy(tail_hbm, ttail)
            pltpu.sync_copy(ttail, out_hbm.at[pl.ds(nblk * 32, tail_rows)])

    return format_kernel(tableT, tail_pk)


def _sc_gather(table_lin, idx2d, n_rows, d):
    nc, ns = _sc_info()
    nw = nc * ns
    chunks_total = n_rows // _GCHUNK
    chunks_per_w = chunks_total // nw
    rows_per_w = chunks_per_w * _GCHUNK

    mesh = plsc.VectorSubcoreMesh(core_axis_name="c", subcore_axis_name="s")

    nq = chunks_per_w // 4

    @functools.partial(
        pl.kernel,
        out_type=jax.ShapeDtypeStruct((n_rows, d), jnp.float32),
        mesh=mesh,
        scratch_types=[
            pltpu.VMEM((chunks_per_w, _GCHUNK), jnp.int32),
            pltpu.VMEM((_GCHUNK, d), jnp.float32),
            pltpu.VMEM((_GCHUNK, d), jnp.float32),
            pltpu.VMEM((_GCHUNK, d), jnp.float32),
            pltpu.VMEM((_GCHUNK, d), jnp.float32),
            pltpu.SemaphoreType.DMA,
            pltpu.SemaphoreType.DMA,
            pltpu.SemaphoreType.DMA,
            pltpu.SemaphoreType.DMA,
            pltpu.SemaphoreType.DMA,
            pltpu.SemaphoreType.DMA,
            pltpu.SemaphoreType.DMA,
            pltpu.SemaphoreType.DMA,
        ],
        compiler_params=pltpu.CompilerParams(use_tc_tiling_on_sc=False),
    )
    def gather_kernel(table_hbm, idx_hbm, out_hbm, idx_v, r0, r1, r2, r3,
                      g0, g1, g2, g3, o0, o1, o2, o3):
        wid = lax.axis_index("s") * nc + lax.axis_index("c")
        cbase = wid * chunks_per_w
        rbase = wid * rows_per_w
        pltpu.sync_copy(idx_hbm.at[pl.ds(cbase, chunks_per_w)], idx_v)
        rows = (r0, r1, r2, r3)
        sgs = (g0, g1, g2, g3)
        sos = (o0, o1, o2, o3)

        def gsrc(j):
            return table_hbm.at[idx_v.at[j]]

        def odst(j):
            return out_hbm.at[pl.ds(rbase + j * _GCHUNK, _GCHUNK)]

        def issue_g(j, p):
            pltpu.async_copy(gsrc(j), rows[p], sgs[p])

        def wait_g(j, p):
            pltpu.make_async_copy(gsrc(j), rows[p], sgs[p]).wait()

        def issue_o(j, p):
            pltpu.async_copy(rows[p], odst(j), sos[p])

        def wait_o(j, p):
            pltpu.make_async_copy(rows[p], odst(j), sos[p]).wait()

        issue_g(0, 0)
        issue_g(1, 1)
        wait_g(0, 0)
        issue_o(0, 0)
        issue_g(2, 2)
        wait_g(1, 1)
        issue_o(1, 1)
        issue_g(3, 3)
        wait_g(2, 2)
        issue_o(2, 2)
        wait_o(0, 0)
        issue_g(4, 0)
        wait_g(3, 3)
        issue_o(3, 3)
        wait_o(1, 1)
        issue_g(5, 1)

        def body(jj, carry):
            for q in range(4):
                s = 4 * jj + q
                p = q
                wait_g(s, p)
                issue_o(s, p)
                wait_o(s - 2, (q + 2) % 4)

                @pl.when(s + 2 < chunks_per_w)
                def _():
                    issue_g(s + 2, (q + 2) % 4)

            return carry

        lax.fori_loop(1, nq, body, 0)
        wait_o(chunks_per_w - 2, 2)
        wait_o(chunks_per_w - 1, 3)

    return gather_kernel(table_lin, idx2d)


def _tc_mlp(flat, W1, b1r, W2r, b2r):
    b_, k = flat.shape
    h = W1.shape[1]
    bm = 1024

    def body(x_ref, w1_ref, b1_ref, w2_ref, b2_ref, o_ref):
        x = x_ref[...]
        hh = jnp.maximum(
            jnp.dot(x, w1_ref[...], preferred_element_type=jnp.float32)
            + b1_ref[...],
            0.0,
        )
        o_ref[...] = jnp.sum(hh * w2_ref[...], axis=1, keepdims=True) + b2_ref[...]

    return pl.pallas_call(
        body,
        grid=(b_ // bm,),
        in_specs=[
            pl.BlockSpec((bm, k), lambda i: (i, 0)),
            pl.BlockSpec((k, h), lambda i: (0, 0)),
            pl.BlockSpec((1, h), lambda i: (0, 0)),
            pl.BlockSpec((1, h), lambda i: (0, 0)),
            pl.BlockSpec((1, 1), lambda i: (0, 0)),
        ],
        out_specs=pl.BlockSpec((bm, 1), lambda i: (i, 0)),
        out_shape=jax.ShapeDtypeStruct((b_, 1), jnp.float32),
    )(flat, W1, b1r, W2r, b2r)


def kernel(X, table, W1, b1, W2, b2):
    b_, f = X.shape
    v, d = table.shape
    h = W1.shape[1]
    n_rows = b_ * f

    tail = v % _GCHUNK
    tail_pk = table[v - tail :, :].reshape(tail * d // _GCHUNK, _GCHUNK)
    table_pk = _sc_format(table.T, tail_pk, v, d)
    table_lin = table_pk.reshape(v, d)

    idx2d = X.reshape(n_rows // _GCHUNK, _GCHUNK)
    rows = _sc_gather(table_lin, idx2d, n_rows, d)
    flat = rows.reshape(b_, f * d)
    return _tc_mlp(flat, W1, b1.reshape(1, h), W2.reshape(1, h), b2.reshape(1, 1))


# speedup vs baseline: 6.2022x; 1.1204x over previous
import functools

import jax
import jax.numpy as jnp
from jax import lax
from jax.experimental import pallas as pl
from jax.experimental.pallas import tpu as pltpu
from jax.experimental.pallas import tpu_sc as plsc


def _sc_info():
    try:
        info = plsc.get_sparse_core_info()
        return info.num_cores, info.num_subcores
    except Exception:
        return 2, 16


_GCHUNK = 128


def _sc_format(tableT, tail_pk, v, d):
    nc, ns = _sc_info()
    nw = nc * ns
    nblk = v // _GCHUNK
    tail_rows = tail_pk.shape[0]
    out_rows = (v * d) // _GCHUNK

    m = 2
    sb_per_w = nblk // (nw * m)
    nleft = nblk - sb_per_w * nw * m
    scols = m * _GCHUNK
    srows = m * d

    mesh = plsc.VectorSubcoreMesh(core_axis_name="c", subcore_axis_name="s")

    @functools.partial(
        pl.kernel,
        out_type=jax.ShapeDtypeStruct((out_rows, _GCHUNK), jnp.float32),
        mesh=mesh,
        scratch_types=[
            pltpu.VMEM((d, scols), jnp.float32),
            pltpu.VMEM((d, scols), jnp.float32),
            pltpu.VMEM((srows, _GCHUNK), jnp.float32),
            pltpu.VMEM((srows, _GCHUNK), jnp.float32),
            pltpu.VMEM((d, scols), jnp.float32),
            pltpu.VMEM((tail_rows, _GCHUNK), jnp.float32),
            pltpu.SemaphoreType.DMA,
            pltpu.SemaphoreType.DMA,
            pltpu.SemaphoreType.DMA,
            pltpu.SemaphoreType.DMA,
        ],
        compiler_params=pltpu.CompilerParams(
            use_tc_tiling_on_sc=True, needs_layout_passes=False
        ),
    )
    def format_kernel(tableT_hbm, tail_hbm, out_hbm, tin0, tin1, tout0, tout1,
                      tskew, ttail, si0, si1, so0, so1):
        wid = lax.axis_index("s") * nc + lax.axis_index("c")
        sb0 = wid * sb_per_w
        d_lo = lax.iota(jnp.int32, 16)
        d_hi = d_lo + 16
        tins = (tin0, tin1)
        touts = (tout0, tout1)
        sis = (si0, si1)
        sos = (so0, so1)

        def src(j):
            return tableT_hbm.at[:, pl.ds((sb0 + j) * scols, scols)]

        def dst(j):
            return out_hbm.at[pl.ds((sb0 + j) * srows, srows)]

        def issue_in(j, p):
            pltpu.async_copy(src(j), tins[p], sis[p])

        def wait_in(j, p):
            pltpu.make_async_copy(src(j), tins[p], sis[p]).wait()

        def issue_out(j, p):
            pltpu.async_copy(touts[p], dst(j), sos[p])

        def wait_out(j, p):
            pltpu.make_async_copy(touts[p], dst(j), sos[p]).wait()

        def compute(tin, tout):
            @plsc.parallel_loop(0, d, unroll=2)
            def _(dd):
                rot = (d_lo + dd) & 15
                dsplat = jnp.zeros((16,), jnp.int32) + dd
                for grp in range(scols // 16):
                    idxa = rot + (16 * grp)
                    vals = plsc.load_gather(tin, [dsplat, idxa])
                    tskew[dd, pl.ds(16 * grp, 16)] = vals

            @plsc.parallel_loop(0, srows, unroll=2)
            def _(u):
                c0 = 4 * u
                cg0 = (c0 >> 4) << 4
                base = (jnp.zeros((16,), jnp.int32) + (c0 - cg0 + 32)) - d_lo
                for q in range(4):
                    idxb = ((base + q) & 15) + cg0
                    for h in range(2):
                        dvec = d_lo if h == 0 else d_hi
                        vals = plsc.load_gather(tskew, [dvec, idxb])
                        tout[u, pl.ds(q * 32 + 16 * h, 16)] = vals

        def steady(j, p, first):
            wait_in(j, p)
            if not first:
                wait_out(j - 2, p)
            compute(tins[p], touts[p])
            issue_out(j, p)
            if j + 2 < sb_per_w:
                issue_in(j + 2, p)

        issue_in(0, 0)
        issue_in(1, 1)
        steady(0, 0, True)
        steady(1, 1, True)

        def body(jj, carry):
            j = 2 + 2 * jj
            wait_in(j, 0)
            wait_out(j - 2, 0)
            compute(tin0, tout0)
            issue_out(j, 0)

            @pl.when(j + 2 < sb_per_w)
            def _():
                pltpu.async_copy(
                    tableT_hbm.at[:, pl.ds((sb0 + j + 2) * scols, scols)],
                    tin0, si0,
                )
            wait_in(j + 1, 1)
            wait_out(j - 1, 1)
            compute(tin1, tout1)
            issue_out(j + 1, 1)

            @pl.when(j + 3 < sb_per_w)
            def _():
                pltpu.async_copy(
                    tableT_hbm.at[:, pl.ds((sb0 + j + 3) * scols, scols)],
                    tin1, si1,
                )

            return carry

        if sb_per_w > 2:
            nst = (sb_per_w - 2) // 2
            lax.fori_loop(0, nst, body, 0)
            for j in range(2 + 2 * nst, sb_per_w):
                p = j % 2
                wait_in(j, p)
                wait_out(j - 2, p)
                compute(tins[p], touts[p])
                issue_out(j, p)
        wait_out(sb_per_w - 2, sb_per_w % 2)
        wait_out(sb_per_w - 1, (sb_per_w - 1) % 2)

        if nleft:
            @pl.when(wid < nleft)
            def _():
                g = nblk - nleft + wid
                pltpu.sync_copy(
                    tableT_hbm.at[:, pl.ds(g * _GCHUNK, _GCHUNK)],
                    tin0.at[:, pl.ds(0, _GCHUNK)],
                )
                def rowpair(rr, carry):
                    for i in range(2):
                        for k in range(8):
                            dvec = d_lo if (k % 2 == 0) else d_hi
                            cvec = jnp.zeros((16,), jnp.int32) + (
                                4 * (2 * rr + i) + (k // 2)
                            )
                            vals = plsc.load_gather(tin0, [dvec, cvec])
                            tout0[2 * rr + i, pl.ds(16 * k, 16)] = vals
                    return carry

                lax.fori_loop(0, 16, rowpair, 0)
                pltpu.sync_copy(
                    tout0.at[pl.ds(0, 32)], out_hbm.at[pl.ds(g * 32, 32)]
                )

        @pl.when(wid == 0)
        def _():
            pltpu.sync_copy(tail_hbm, ttail)
            pltpu.sync_copy(ttail, out_hbm.at[pl.ds(nblk * 32, tail_rows)])

    return format_kernel(tableT, tail_pk)


def _sc_gather(table_lin, idx2d, n_rows, d):
    nc, ns = _sc_info()
    nw = nc * ns
    chunks_total = n_rows // _GCHUNK
    chunks_per_w = chunks_total // nw
    rows_per_w = chunks_per_w * _GCHUNK

    mesh = plsc.VectorSubcoreMesh(core_axis_name="c", subcore_axis_name="s")

    nq = chunks_per_w // 4

    @functools.partial(
        pl.kernel,
        out_type=jax.ShapeDtypeStruct((n_rows, d), jnp.float32),
        mesh=mesh,
        scratch_types=[
            pltpu.VMEM((chunks_per_w, _GCHUNK), jnp.int32),
            pltpu.VMEM((_GCHUNK, d), jnp.float32),
            pltpu.VMEM((_GCHUNK, d), jnp.float32),
            pltpu.VMEM((_GCHUNK, d), jnp.float32),
            pltpu.VMEM((_GCHUNK, d), jnp.float32),
            pltpu.SemaphoreType.DMA,
            pltpu.SemaphoreType.DMA,
            pltpu.SemaphoreType.DMA,
            pltpu.SemaphoreType.DMA,
            pltpu.SemaphoreType.DMA,
            pltpu.SemaphoreType.DMA,
            pltpu.SemaphoreType.DMA,
            pltpu.SemaphoreType.DMA,
        ],
        compiler_params=pltpu.CompilerParams(use_tc_tiling_on_sc=False),
    )
    def gather_kernel(table_hbm, idx_hbm, out_hbm, idx_v, r0, r1, r2, r3,
                      g0, g1, g2, g3, o0, o1, o2, o3):
        wid = lax.axis_index("s") * nc + lax.axis_index("c")
        cbase = wid * chunks_per_w
        rbase = wid * rows_per_w
        pltpu.sync_copy(idx_hbm.at[pl.ds(cbase, chunks_per_w)], idx_v)
        rows = (r0, r1, r2, r3)
        sgs = (g0, g1, g2, g3)
        sos = (o0, o1, o2, o3)

        def gsrc(j):
            return table_hbm.at[idx_v.at[j]]

        def odst(j):
            return out_hbm.at[pl.ds(rbase + j * _GCHUNK, _GCHUNK)]

        def issue_g(j, p):
            pltpu.async_copy(gsrc(j), rows[p], sgs[p])

        def wait_g(j, p):
            pltpu.make_async_copy(gsrc(j), rows[p], sgs[p]).wait()

        def issue_o(j, p):
            pltpu.async_copy(rows[p], odst(j), sos[p])

        def wait_o(j, p):
            pltpu.make_async_copy(rows[p], odst(j), sos[p]).wait()

        issue_g(0, 0)
        issue_g(1, 1)
        wait_g(0, 0)
        issue_o(0, 0)
        issue_g(2, 2)
        wait_g(1, 1)
        issue_o(1, 1)
        issue_g(3, 3)
        wait_g(2, 2)
        issue_o(2, 2)
        wait_o(0, 0)
        issue_g(4, 0)
        wait_g(3, 3)
        issue_o(3, 3)
        wait_o(1, 1)
        issue_g(5, 1)

        def body(jj, carry):
            for q in range(4):
                s = 4 * jj + q
                p = q
                wait_g(s, p)
                issue_o(s, p)
                wait_o(s - 2, (q + 2) % 4)

                @pl.when(s + 2 < chunks_per_w)
                def _():
                    issue_g(s + 2, (q + 2) % 4)

            return carry

        lax.fori_loop(1, nq, body, 0)
        wait_o(chunks_per_w - 2, 2)
        wait_o(chunks_per_w - 1, 3)

    return gather_kernel(table_lin, idx2d)


def _tc_mlp(flat3, W1r, b1r, W2r, b2r):
    b_, p_, _ = flat3.shape
    h = W1r.shape[2]
    bm = 1024

    def body(x_ref, w1_ref, b1_ref, w2_ref, b2_ref, o_ref):
        acc = jnp.dot(
            x_ref[:, 0, :], w1_ref[0], preferred_element_type=jnp.float32
        )
        for j in range(1, p_):
            acc += jnp.dot(
                x_ref[:, j, :], w1_ref[j], preferred_element_type=jnp.float32
            )
        hh = jnp.maximum(acc + b1_ref[...], 0.0)
        o_ref[...] = jnp.sum(hh * w2_ref[...], axis=1, keepdims=True) + b2_ref[...]

    return pl.pallas_call(
        body,
        grid=(b_ // bm,),
        in_specs=[
            pl.BlockSpec((bm, p_, _GCHUNK), lambda i: (i, 0, 0)),
            pl.BlockSpec((p_, _GCHUNK, h), lambda i: (0, 0, 0)),
            pl.BlockSpec((1, h), lambda i: (0, 0)),
            pl.BlockSpec((1, h), lambda i: (0, 0)),
            pl.BlockSpec((1, 1), lambda i: (0, 0)),
        ],
        out_specs=pl.BlockSpec((bm, 1), lambda i: (i, 0)),
        out_shape=jax.ShapeDtypeStruct((b_, 1), jnp.float32),
    )(flat3, W1r, b1r, W2r, b2r)


def kernel(X, table, W1, b1, W2, b2):
    b_, f = X.shape
    v, d = table.shape
    h = W1.shape[1]

    tail = v % _GCHUNK
    tail_pk = table[v - tail :, :].reshape(tail * d // _GCHUNK, _GCHUNK)
    table_pk = _sc_format(table.T, tail_pk, v, d)
    table_lin = table_pk.reshape(v, d)

    fp = 32 * ((f + 31) // 32)
    npad = fp - f
    pad_idx = (
        jnp.arange(b_, dtype=jnp.int32)[:, None] * 61
        + jnp.arange(npad, dtype=jnp.int32)[None, :] * 9973
    ) % v
    Xp = jnp.concatenate([X, pad_idx], axis=1)
    n_rows = b_ * fp
    idx2d = Xp.reshape(n_rows // _GCHUNK, _GCHUNK)

    rows = _sc_gather(table_lin, idx2d, n_rows, d)
    flat3 = rows.reshape(b_, fp * d // _GCHUNK, _GCHUNK)
    W1r = jnp.pad(
        W1.reshape(f, d, h), ((0, npad), (0, 0), (0, 0))
    ).reshape(fp * d // _GCHUNK, _GCHUNK, h)
    return _tc_mlp(flat3, W1r, b1.reshape(1, h), W2.reshape(1, h), b2.reshape(1, 1))
